# baseline (speedup 1.0000x reference)
import functools

import jax
import jax.numpy as jnp
from jax.experimental import pallas as pl
from jax.experimental.pallas import tpu as pltpu
from jax.experimental.pallas import tpu_sc as plsc

_VS = 32
_NB = _VS * _VS * _VS
_B = 16
_N = 262144
_NCOL = 384
_ROWS = _N * 3 // _NCOL
_RBLK = 256
_KSTEPS = _ROWS // _RBLK


def _minmax_kernel(x_ref, mn_ref, mx_ref):
    j = pl.program_id(1)
    x = x_ref[0]
    cmn = jnp.min(x, axis=0, keepdims=True)
    cmx = jnp.max(x, axis=0, keepdims=True)

    @pl.when(j == 0)
    def _():
        mn_ref[0] = cmn
        mx_ref[0] = cmx

    @pl.when(j > 0)
    def _():
        mn_ref[0] = jnp.minimum(mn_ref[0], cmn)
        mx_ref[0] = jnp.maximum(mx_ref[0], cmx)

    @pl.when(j == _KSTEPS - 1)
    def _():
        col = jax.lax.broadcasted_iota(jnp.int32, (1, _NCOL), 1) % 3
        mnv, mxv = mn_ref[0], mx_ref[0]
        big = jnp.float32(jnp.inf)
        mn0 = jnp.min(jnp.where(col == 0, mnv, big))
        mn1 = jnp.min(jnp.where(col == 1, mnv, big))
        mn2 = jnp.min(jnp.where(col == 2, mnv, big))
        mx0 = jnp.max(jnp.where(col == 0, mxv, -big))
        mx1 = jnp.max(jnp.where(col == 1, mxv, -big))
        mx2 = jnp.max(jnp.where(col == 2, mxv, -big))
        mn_ref[0] = jnp.where(col == 0, mn0, jnp.where(col == 1, mn1, mn2))
        mx_ref[0] = jnp.where(col == 0, mx0, jnp.where(col == 1, mx1, mx2))


def _lin_kernel(x_ref, mn_ref, mx_ref, s_ref, out_ref):
    x = x_ref[0]
    mn = mn_ref[0]
    mx = mx_ref[0]
    t = (x - mn) / (mx - mn) * jnp.float32(_VS - 1)
    col = jax.lax.broadcasted_iota(jnp.int32, (1, _NCOL), 1) % 3
    w = jnp.where(col == 0, jnp.float32(_VS * _VS),
                  jnp.where(col == 1, jnp.float32(_VS), jnp.float32(1.0)))
    e = jnp.floor(t) * w
    lin = jax.lax.dot_general(e, s_ref[...], (((1,), (0,)), ((), ())),
                              preferred_element_type=jnp.float32)
    out_ref[0] = lin.astype(jnp.int32)


_CH = 4096
_GROUPS = _CH // 16


def _hist_body(lin_hbm, out_hbm, hist, buf, sem):
    del sem
    c = jax.lax.axis_index("c")
    s = jax.lax.axis_index("s")
    wid = c * 16 + s

    @pl.when(wid < _B)
    def _():
        def zbody(i, carry):
            hist[pl.ds(i * 16, 16)] = jnp.zeros((16,), jnp.int32)
            return carry
        jax.lax.fori_loop(0, _NB // 16, zbody, 0)

        ones = jnp.ones((16,), jnp.int32)

        def chunk_body(k, carry):
            pltpu.sync_copy(lin_hbm.at[wid, pl.ds(k * _CH, _CH)], buf)

            def gbody(g, carry2):
                v = buf[pl.ds(g * 16, 16)]
                plsc.addupdate_scatter(hist, [v], ones)
                return carry2
            jax.lax.fori_loop(0, _GROUPS, gbody, 0)
            return carry
        jax.lax.fori_loop(0, _N // _CH, chunk_body, 0)
        pltpu.sync_copy(hist, out_hbm.at[wid])


@functools.lru_cache(maxsize=None)
def _make_hist_sc():
    return pl.kernel(
        _hist_body,
        out_type=jax.ShapeDtypeStruct((_B, _NB), jnp.int32),
        mesh=plsc.VectorSubcoreMesh(core_axis_name="c", subcore_axis_name="s"),
        compiler_params=pltpu.CompilerParams(needs_layout_passes=False),
        scratch_types=[
            pltpu.VMEM((_NB,), jnp.int32),
            pltpu.VMEM((_CH,), jnp.int32),
            pltpu.SemaphoreType.DMA,
        ],
    )


def kernel(point_cloud):
    pts = point_cloud.reshape(_B, _ROWS, _NCOL)
    mn, mx = pl.pallas_call(
        _minmax_kernel,
        grid=(_B, _KSTEPS),
        in_specs=[pl.BlockSpec((1, _RBLK, _NCOL), lambda b, j: (b, j, 0))],
        out_specs=[
            pl.BlockSpec((1, 1, _NCOL), lambda b, j: (b, 0, 0)),
            pl.BlockSpec((1, 1, _NCOL), lambda b, j: (b, 0, 0)),
        ],
        out_shape=[
            jax.ShapeDtypeStruct((_B, 1, _NCOL), jnp.float32),
            jax.ShapeDtypeStruct((_B, 1, _NCOL), jnp.float32),
        ],
    )(pts)

    sel = jnp.repeat(jnp.eye(128, dtype=jnp.float32), 3, axis=0)
    lin3 = pl.pallas_call(
        _lin_kernel,
        grid=(_B, _KSTEPS),
        in_specs=[
            pl.BlockSpec((1, _RBLK, _NCOL), lambda b, j: (b, j, 0)),
            pl.BlockSpec((1, 1, _NCOL), lambda b, j: (b, 0, 0)),
            pl.BlockSpec((1, 1, _NCOL), lambda b, j: (b, 0, 0)),
            pl.BlockSpec((_NCOL, 128), lambda b, j: (0, 0)),
        ],
        out_specs=pl.BlockSpec((1, _RBLK, 128), lambda b, j: (b, j, 0)),
        out_shape=jax.ShapeDtypeStruct((_B, _ROWS, 128), jnp.int32),
    )(pts, mn, mx, sel)

    lin = lin3.reshape(_B, _N)
    counts = _make_hist_sc()(lin)
    return lin, counts


# speedup vs baseline: 4.9961x; 4.9961x over previous
import functools

import jax
import jax.numpy as jnp
from jax.experimental import pallas as pl
from jax.experimental.pallas import tpu as pltpu
from jax.experimental.pallas import tpu_sc as plsc

_VS = 32
_NB = _VS * _VS * _VS
_B = 16
_N = 262144
_R = 48

_CHA = 8192
_KA = _N // _CHA
_CHB = 8192
_KB = _N // _CHB


def _minmax_kernel(x_ref, mn_ref, sc_ref, mx_acc):
    k = pl.program_id(0)
    x = x_ref[...]
    bmn = jnp.broadcast_to(jnp.min(x, axis=1, keepdims=True), (_R, 128))
    bmx = jnp.broadcast_to(jnp.max(x, axis=1, keepdims=True), (_R, 128))

    @pl.when(k == 0)
    def _():
        mn_ref[...] = bmn
        mx_acc[...] = bmx

    @pl.when(k > 0)
    def _():
        mn_ref[...] = jnp.minimum(mn_ref[...], bmn)
        mx_acc[...] = jnp.maximum(mx_acc[...], bmx)

    @pl.when(k == _KA - 1)
    def _():
        sc_ref[...] = jnp.float32(_VS - 1) / (mx_acc[...] - mn_ref[...])


def _lin_kernel(x_ref, mn_ref, sc_ref, out_ref):
    x = x_ref[...]
    mnb = jnp.broadcast_to(mn_ref[:, :1], (_R, _CHB))
    scb = jnp.broadcast_to(sc_ref[:, :1], (_R, _CHB))
    f = jnp.floor((x - mnb) * scb)
    lin = (f[0:16] * jnp.float32(_VS * _VS) + f[16:32] * jnp.float32(_VS)
           + f[32:48])
    out_ref[...] = lin.astype(jnp.int32)


_CH = 4096
_GROUPS = _CH // 16


def _hist_body(lin_hbm, out_hbm, hist, buf, sem):
    del sem
    c = jax.lax.axis_index("c")
    s = jax.lax.axis_index("s")
    wid = c * 16 + s

    @pl.when(wid < _B)
    def _():
        def zbody(i, carry):
            hist[pl.ds(i * 16, 16)] = jnp.zeros((16,), jnp.int32)
            return carry
        jax.lax.fori_loop(0, _NB // 16, zbody, 0)

        ones = jnp.ones((16,), jnp.int32)

        def chunk_body(k, carry):
            pltpu.sync_copy(lin_hbm.at[wid, pl.ds(k * _CH, _CH)], buf)

            def gbody(g, carry2):
                v = buf[pl.ds(g * 16, 16)]
                plsc.addupdate_scatter(hist, [v], ones)
                return carry2
            jax.lax.fori_loop(0, _GROUPS, gbody, 0)
            return carry
        jax.lax.fori_loop(0, _N // _CH, chunk_body, 0)
        pltpu.sync_copy(hist, out_hbm.at[wid])


@functools.lru_cache(maxsize=None)
def _make_hist_sc():
    return pl.kernel(
        _hist_body,
        out_type=jax.ShapeDtypeStruct((_B, _NB), jnp.int32),
        mesh=plsc.VectorSubcoreMesh(core_axis_name="c", subcore_axis_name="s"),
        compiler_params=pltpu.CompilerParams(needs_layout_passes=False),
        scratch_types=[
            pltpu.VMEM((_NB,), jnp.int32),
            pltpu.VMEM((_CH,), jnp.int32),
            pltpu.SemaphoreType.DMA,
        ],
    )


def kernel(point_cloud):
    x2 = point_cloud.transpose(2, 0, 1).reshape(_R, _N)
    mn, sc = pl.pallas_call(
        _minmax_kernel,
        grid=(_KA,),
        in_specs=[pl.BlockSpec((_R, _CHA), lambda k: (0, k))],
        out_specs=[
            pl.BlockSpec((_R, 128), lambda k: (0, 0)),
            pl.BlockSpec((_R, 128), lambda k: (0, 0)),
        ],
        out_shape=[
            jax.ShapeDtypeStruct((_R, 128), jnp.float32),
            jax.ShapeDtypeStruct((_R, 128), jnp.float32),
        ],
        scratch_shapes=[pltpu.VMEM((_R, 128), jnp.float32)],
    )(x2)

    lin = pl.pallas_call(
        _lin_kernel,
        grid=(_KB,),
        in_specs=[
            pl.BlockSpec((_R, _CHB), lambda k: (0, k)),
            pl.BlockSpec((_R, 128), lambda k: (0, 0)),
            pl.BlockSpec((_R, 128), lambda k: (0, 0)),
        ],
        out_specs=pl.BlockSpec((_B, _CHB), lambda k: (0, k)),
        out_shape=jax.ShapeDtypeStruct((_B, _N), jnp.int32),
    )(x2, mn, sc)

    counts = _make_hist_sc()(lin)
    return lin, counts


# speedup vs baseline: 6.4668x; 1.2944x over previous
import functools

import jax
import jax.numpy as jnp
from jax.experimental import pallas as pl
from jax.experimental.pallas import tpu as pltpu
from jax.experimental.pallas import tpu_sc as plsc

_VS = 32
_NB = _VS * _VS * _VS
_B = 16
_N = 262144
_R = 48

_CHA = 8192
_KA = _N // _CHA
_CHB = 8192
_KB = _N // _CHB


def _minmax_kernel(x_ref, mn_ref, sc_ref, mx_acc):
    k = pl.program_id(0)
    x = x_ref[...]
    bmn = jnp.broadcast_to(jnp.min(x, axis=1, keepdims=True), (_R, 128))
    bmx = jnp.broadcast_to(jnp.max(x, axis=1, keepdims=True), (_R, 128))

    @pl.when(k == 0)
    def _():
        mn_ref[...] = bmn
        mx_acc[...] = bmx

    @pl.when(k > 0)
    def _():
        mn_ref[...] = jnp.minimum(mn_ref[...], bmn)
        mx_acc[...] = jnp.maximum(mx_acc[...], bmx)

    @pl.when(k == _KA - 1)
    def _():
        sc_ref[...] = jnp.float32(_VS - 1) / (mx_acc[...] - mn_ref[...])


def _lin_kernel(x_ref, mn_ref, sc_ref, out_ref):
    x = x_ref[...]
    mnb = jnp.broadcast_to(mn_ref[:, :1], (_R, _CHB))
    scb = jnp.broadcast_to(sc_ref[:, :1], (_R, _CHB))
    f = jnp.floor((x - mnb) * scb)
    lin = (f[0:16] * jnp.float32(_VS * _VS) + f[16:32] * jnp.float32(_VS)
           + f[32:48])
    out_ref[...] = lin.astype(jnp.int32)


_CH = 8192
_GROUPS = _CH // 16
_NCK = _N // _CH


def _hist_body(lin_hbm, out_hbm, hist, buf0, buf1, sem0, sem1):
    c = jax.lax.axis_index("c")
    s = jax.lax.axis_index("s")
    wid = c * 16 + s

    @pl.when(wid < _B)
    def _():
        def zbody(i, carry):
            hist[pl.ds(i * 16, 16)] = jnp.zeros((16,), jnp.int32)
            return carry
        jax.lax.fori_loop(0, _NB // 16, zbody, 0, unroll=16)

        ones = jnp.ones((16,), jnp.int32)
        bufs = (buf0, buf1)
        sems = (sem0, sem1)
        descs = {0: pltpu.async_copy(lin_hbm.at[wid, pl.ds(0, _CH)],
                                     buf0, sem0)}
        for k in range(_NCK):
            descs.pop(k).wait()
            if k + 1 < _NCK:
                descs[k + 1] = pltpu.async_copy(
                    lin_hbm.at[wid, pl.ds((k + 1) * _CH, _CH)],
                    bufs[(k + 1) % 2], sems[(k + 1) % 2])
            cur = bufs[k % 2]

            def gbody(g, carry2, cur=cur):
                v = cur[pl.ds(g * 16, 16)]
                plsc.addupdate_scatter(hist, [v], ones)
                return carry2
            jax.lax.fori_loop(0, _GROUPS, gbody, 0, unroll=8)
        pltpu.sync_copy(hist, out_hbm.at[wid])


@functools.lru_cache(maxsize=None)
def _make_hist_sc():
    return pl.kernel(
        _hist_body,
        out_type=jax.ShapeDtypeStruct((_B, _NB), jnp.int32),
        mesh=plsc.VectorSubcoreMesh(core_axis_name="c", subcore_axis_name="s"),
        compiler_params=pltpu.CompilerParams(needs_layout_passes=False),
        scratch_types=[
            pltpu.VMEM((_NB,), jnp.int32),
            pltpu.VMEM((_CH,), jnp.int32),
            pltpu.VMEM((_CH,), jnp.int32),
            pltpu.SemaphoreType.DMA,
            pltpu.SemaphoreType.DMA,
        ],
    )


def kernel(point_cloud):
    x2 = point_cloud.transpose(2, 0, 1).reshape(_R, _N)
    mn, sc = pl.pallas_call(
        _minmax_kernel,
        grid=(_KA,),
        in_specs=[pl.BlockSpec((_R, _CHA), lambda k: (0, k))],
        out_specs=[
            pl.BlockSpec((_R, 128), lambda k: (0, 0)),
            pl.BlockSpec((_R, 128), lambda k: (0, 0)),
        ],
        out_shape=[
            jax.ShapeDtypeStruct((_R, 128), jnp.float32),
            jax.ShapeDtypeStruct((_R, 128), jnp.float32),
        ],
        scratch_shapes=[pltpu.VMEM((_R, 128), jnp.float32)],
    )(x2)

    lin = pl.pallas_call(
        _lin_kernel,
        grid=(_KB,),
        in_specs=[
            pl.BlockSpec((_R, _CHB), lambda k: (0, k)),
            pl.BlockSpec((_R, 128), lambda k: (0, 0)),
            pl.BlockSpec((_R, 128), lambda k: (0, 0)),
        ],
        out_specs=pl.BlockSpec((_B, _CHB), lambda k: (0, k)),
        out_shape=jax.ShapeDtypeStruct((_B, _N), jnp.int32),
    )(x2, mn, sc)

    counts = _make_hist_sc()(lin)
    return lin, counts


# speedup vs baseline: 8.2483x; 1.2755x over previous
import functools

import jax
import jax.numpy as jnp
from jax.experimental import pallas as pl
from jax.experimental.pallas import tpu as pltpu
from jax.experimental.pallas import tpu_sc as plsc

_VS = 32
_NB = _VS * _VS * _VS
_B = 16
_N = 262144
_R = 48

_CHA = 8192
_KA = _N // _CHA
_CHB = 8192
_KB = _N // _CHB


def _minmax_kernel(x_ref, mn_ref, sc_ref, mx_acc):
    k = pl.program_id(0)
    x = x_ref[...]
    bmn = jnp.broadcast_to(jnp.min(x, axis=1, keepdims=True), (_R, 128))
    bmx = jnp.broadcast_to(jnp.max(x, axis=1, keepdims=True), (_R, 128))

    @pl.when(k == 0)
    def _():
        mn_ref[...] = bmn
        mx_acc[...] = bmx

    @pl.when(k > 0)
    def _():
        mn_ref[...] = jnp.minimum(mn_ref[...], bmn)
        mx_acc[...] = jnp.maximum(mx_acc[...], bmx)

    @pl.when(k == _KA - 1)
    def _():
        sc_ref[...] = jnp.float32(_VS - 1) / (mx_acc[...] - mn_ref[...])


def _lin_kernel(x_ref, mn_ref, sc_ref, out_ref):
    x = x_ref[...]
    mnb = jnp.broadcast_to(mn_ref[:, :1], (_R, _CHB))
    scb = jnp.broadcast_to(sc_ref[:, :1], (_R, _CHB))
    f = jnp.floor((x - mnb) * scb)
    lin = (f[0:16] * jnp.float32(_VS * _VS) + f[16:32] * jnp.float32(_VS)
           + f[32:48])
    out_ref[...] = lin.astype(jnp.int32)


_CH = 8192
_GROUPS = _CH // 16
_HALF = _N // 2
_NCK = _HALF // _CH


def _hist_body(lin_hbm, out_hbm, hist, mbuf, shared, buf0, buf1, sem0, sem1):
    c = jax.lax.axis_index("c")
    s = jax.lax.axis_index("s")
    batch = c * 8 + s // 2
    half = s % 2
    base = half * _HALF

    def zbody(i, carry):
        hist[pl.ds(i * 16, 16)] = jnp.zeros((16,), jnp.int32)
        return carry
    jax.lax.fori_loop(0, _NB // 16, zbody, 0, unroll=16)

    ones = jnp.ones((16,), jnp.int32)
    bufs = (buf0, buf1)
    sems = (sem0, sem1)
    descs = {0: pltpu.async_copy(lin_hbm.at[batch, pl.ds(base, _CH)],
                                 buf0, sem0)}
    for k in range(_NCK):
        descs.pop(k).wait()
        if k + 1 < _NCK:
            descs[k + 1] = pltpu.async_copy(
                lin_hbm.at[batch, pl.ds(base + (k + 1) * _CH, _CH)],
                bufs[(k + 1) % 2], sems[(k + 1) % 2])
        cur = bufs[k % 2]

        def gbody(g, carry2, cur=cur):
            v = cur[pl.ds(g * 16, 16)]
            plsc.addupdate_scatter(hist, [v], ones)
            return carry2
        jax.lax.fori_loop(0, _GROUPS, gbody, 0, unroll=8)

    @pl.when(half == 1)
    def _():
        pltpu.sync_copy(hist, shared.at[s // 2])

    plsc.subcore_barrier()

    @pl.when(half == 0)
    def _():
        pltpu.sync_copy(shared.at[s // 2], mbuf)

        def abody(i, carry):
            sl = pl.ds(i * 16, 16)
            hist[sl] = hist[sl] + mbuf[sl]
            return carry
        jax.lax.fori_loop(0, _NB // 16, abody, 0, unroll=8)
        pltpu.sync_copy(hist, out_hbm.at[batch])


@functools.lru_cache(maxsize=None)
def _make_hist_sc():
    return pl.kernel(
        _hist_body,
        out_type=jax.ShapeDtypeStruct((_B, _NB), jnp.int32),
        mesh=plsc.VectorSubcoreMesh(core_axis_name="c", subcore_axis_name="s"),
        compiler_params=pltpu.CompilerParams(needs_layout_passes=False),
        scratch_types=[
            pltpu.VMEM((_NB,), jnp.int32),
            pltpu.VMEM((_NB,), jnp.int32),
            pltpu.VMEM_SHARED((8, _NB), jnp.int32),
            pltpu.VMEM((_CH,), jnp.int32),
            pltpu.VMEM((_CH,), jnp.int32),
            pltpu.SemaphoreType.DMA,
            pltpu.SemaphoreType.DMA,
        ],
    )


def kernel(point_cloud):
    x2 = point_cloud.transpose(2, 0, 1).reshape(_R, _N)
    mn, sc = pl.pallas_call(
        _minmax_kernel,
        grid=(_KA,),
        in_specs=[pl.BlockSpec((_R, _CHA), lambda k: (0, k))],
        out_specs=[
            pl.BlockSpec((_R, 128), lambda k: (0, 0)),
            pl.BlockSpec((_R, 128), lambda k: (0, 0)),
        ],
        out_shape=[
            jax.ShapeDtypeStruct((_R, 128), jnp.float32),
            jax.ShapeDtypeStruct((_R, 128), jnp.float32),
        ],
        scratch_shapes=[pltpu.VMEM((_R, 128), jnp.float32)],
    )(x2)

    lin = pl.pallas_call(
        _lin_kernel,
        grid=(_KB,),
        in_specs=[
            pl.BlockSpec((_R, _CHB), lambda k: (0, k)),
            pl.BlockSpec((_R, 128), lambda k: (0, 0)),
            pl.BlockSpec((_R, 128), lambda k: (0, 0)),
        ],
        out_specs=pl.BlockSpec((_B, _CHB), lambda k: (0, k)),
        out_shape=jax.ShapeDtypeStruct((_B, _N), jnp.int32),
    )(x2, mn, sc)

    counts = _make_hist_sc()(lin)
    return lin, counts


# speedup vs baseline: 14.2219x; 1.7242x over previous
import functools

import jax
import jax.numpy as jnp
from jax.experimental import pallas as pl
from jax.experimental.pallas import tpu as pltpu
from jax.experimental.pallas import tpu_sc as plsc

_VS = 32
_NB = _VS * _VS * _VS
_B = 16
_N = 262144
_R = 48

_CHA = 65536
_KA = _N // _CHA
_CHB = 32768
_KB = _N // _CHB


def _minmax_kernel(x_ref, mn_ref, sc_ref, mx_acc):
    k = pl.program_id(0)
    x = x_ref[...]
    bmn = jnp.broadcast_to(jnp.min(x, axis=1, keepdims=True), (_R, 128))
    bmx = jnp.broadcast_to(jnp.max(x, axis=1, keepdims=True), (_R, 128))

    @pl.when(k == 0)
    def _():
        mn_ref[...] = bmn
        mx_acc[...] = bmx

    @pl.when(k > 0)
    def _():
        mn_ref[...] = jnp.minimum(mn_ref[...], bmn)
        mx_acc[...] = jnp.maximum(mx_acc[...], bmx)

    @pl.when(k == _KA - 1)
    def _():
        sc_ref[...] = jnp.float32(_VS - 1) / (mx_acc[...] - mn_ref[...])


def _lin_kernel(x_ref, mn_ref, sc_ref, out_ref):
    x = x_ref[...]
    mnb = jnp.broadcast_to(mn_ref[:, :1], (_R, _CHB))
    scb = jnp.broadcast_to(sc_ref[:, :1], (_R, _CHB))
    f = jnp.floor((x - mnb) * scb)
    lin = (f[0:16] * jnp.float32(_VS * _VS) + f[16:32] * jnp.float32(_VS)
           + f[32:48])
    out_ref[...] = lin.astype(jnp.int32)


_CH = 8192
_GROUPS = _CH // 16
_HALF = _N // 2
_NCK = _HALF // _CH


def _hist_body(lin_hbm, out_hbm, hist, mbuf, shared, buf0, buf1, sem0, sem1):
    c = jax.lax.axis_index("c")
    s = jax.lax.axis_index("s")
    batch = c * 8 + s // 2
    half = s % 2
    base = half * _HALF

    @plsc.parallel_loop(0, _NB // 16, 1, unroll=16)
    def _zero(i):
        hist[pl.ds(i * 16, 16)] = jnp.zeros((16,), jnp.int32)

    ones = jnp.ones((16,), jnp.int32)
    bufs = (buf0, buf1)
    sems = (sem0, sem1)
    descs = {0: pltpu.async_copy(lin_hbm.at[batch, pl.ds(base, _CH)],
                                 buf0, sem0)}
    for k in range(_NCK):
        descs.pop(k).wait()
        if k + 1 < _NCK:
            descs[k + 1] = pltpu.async_copy(
                lin_hbm.at[batch, pl.ds(base + (k + 1) * _CH, _CH)],
                bufs[(k + 1) % 2], sems[(k + 1) % 2])
        cur = bufs[k % 2]

        def gbody(g, cur=cur):
            v = cur[pl.ds(g * 16, 16)]
            plsc.addupdate_scatter(hist, [v], ones)
        plsc.parallel_loop(0, _GROUPS, 1, unroll=8)(gbody)

    @pl.when(half == 1)
    def _():
        pltpu.sync_copy(hist, shared.at[s // 2])

    plsc.subcore_barrier()

    @pl.when(half == 0)
    def _():
        pltpu.sync_copy(shared.at[s // 2], mbuf)

        @plsc.parallel_loop(0, _NB // 16, 1, unroll=8)
        def _madd(i):
            sl = pl.ds(i * 16, 16)
            hist[sl] = hist[sl] + mbuf[sl]
        pltpu.sync_copy(hist, out_hbm.at[batch])


@functools.lru_cache(maxsize=None)
def _make_hist_sc():
    return pl.kernel(
        _hist_body,
        out_type=jax.ShapeDtypeStruct((_B, _NB), jnp.int32),
        mesh=plsc.VectorSubcoreMesh(core_axis_name="c", subcore_axis_name="s"),
        compiler_params=pltpu.CompilerParams(needs_layout_passes=False),
        scratch_types=[
            pltpu.VMEM((_NB,), jnp.int32),
            pltpu.VMEM((_NB,), jnp.int32),
            pltpu.VMEM_SHARED((8, _NB), jnp.int32),
            pltpu.VMEM((_CH,), jnp.int32),
            pltpu.VMEM((_CH,), jnp.int32),
            pltpu.SemaphoreType.DMA,
            pltpu.SemaphoreType.DMA,
        ],
    )


def kernel(point_cloud):
    x2 = point_cloud.transpose(2, 0, 1).reshape(_R, _N)
    mn, sc = pl.pallas_call(
        _minmax_kernel,
        grid=(_KA,),
        in_specs=[pl.BlockSpec((_R, _CHA), lambda k: (0, k))],
        out_specs=[
            pl.BlockSpec((_R, 128), lambda k: (0, 0)),
            pl.BlockSpec((_R, 128), lambda k: (0, 0)),
        ],
        out_shape=[
            jax.ShapeDtypeStruct((_R, 128), jnp.float32),
            jax.ShapeDtypeStruct((_R, 128), jnp.float32),
        ],
        scratch_shapes=[pltpu.VMEM((_R, 128), jnp.float32)],
    )(x2)

    lin = pl.pallas_call(
        _lin_kernel,
        grid=(_KB,),
        in_specs=[
            pl.BlockSpec((_R, _CHB), lambda k: (0, k)),
            pl.BlockSpec((_R, 128), lambda k: (0, 0)),
            pl.BlockSpec((_R, 128), lambda k: (0, 0)),
        ],
        out_specs=pl.BlockSpec((_B, _CHB), lambda k: (0, k)),
        out_shape=jax.ShapeDtypeStruct((_B, _N), jnp.int32),
    )(x2, mn, sc)

    counts = _make_hist_sc()(lin)
    return lin, counts


# speedup vs baseline: 14.4354x; 1.0150x over previous
import functools

import jax
import jax.numpy as jnp
from jax.experimental import pallas as pl
from jax.experimental.pallas import tpu as pltpu
from jax.experimental.pallas import tpu_sc as plsc

_VS = 32
_NB = _VS * _VS * _VS
_B = 16
_N = 262144
_R = 48

_CHA = 65536
_KA = _N // _CHA
_CHB = 32768
_KB = _N // _CHB


def _minmax_kernel(x_ref, mn_ref, sc_ref, mx_acc):
    k = pl.program_id(0)
    x = x_ref[...]
    bmn = jnp.broadcast_to(jnp.min(x, axis=1, keepdims=True), (_R, 128))
    bmx = jnp.broadcast_to(jnp.max(x, axis=1, keepdims=True), (_R, 128))

    @pl.when(k == 0)
    def _():
        mn_ref[...] = bmn
        mx_acc[...] = bmx

    @pl.when(k > 0)
    def _():
        mn_ref[...] = jnp.minimum(mn_ref[...], bmn)
        mx_acc[...] = jnp.maximum(mx_acc[...], bmx)

    @pl.when(k == _KA - 1)
    def _():
        sc_ref[...] = jnp.float32(_VS - 1) / (mx_acc[...] - mn_ref[...])


def _lin_kernel(x_ref, mn_ref, sc_ref, out_ref):
    x = x_ref[...]
    mnb = jnp.broadcast_to(mn_ref[:, :1], (_R, _CHB))
    scb = jnp.broadcast_to(sc_ref[:, :1], (_R, _CHB))
    f = jnp.floor((x - mnb) * scb)
    lin = (f[0:16] * jnp.float32(_VS * _VS) + f[16:32] * jnp.float32(_VS)
           + f[32:48])
    out_ref[...] = lin.astype(jnp.int32)


_CH = 16384
_GROUPS = _CH // 16
_HALF = _N // 2
_NCK = _HALF // _CH


def _hist_body(lin_hbm, out_hbm, hist, mbuf, shared, buf0, buf1, sem0, sem1):
    c = jax.lax.axis_index("c")
    s = jax.lax.axis_index("s")
    batch = c * 8 + s // 2
    half = s % 2
    base = half * _HALF

    @plsc.parallel_loop(0, _NB // 16, 1, unroll=16)
    def _zero(i):
        hist[pl.ds(i * 16, 16)] = jnp.zeros((16,), jnp.int32)

    ones = jnp.ones((16,), jnp.int32)
    bufs = (buf0, buf1)
    sems = (sem0, sem1)
    descs = {0: pltpu.async_copy(lin_hbm.at[batch, pl.ds(base, _CH)],
                                 buf0, sem0)}
    for k in range(_NCK):
        descs.pop(k).wait()
        if k + 1 < _NCK:
            descs[k + 1] = pltpu.async_copy(
                lin_hbm.at[batch, pl.ds(base + (k + 1) * _CH, _CH)],
                bufs[(k + 1) % 2], sems[(k + 1) % 2])
        cur = bufs[k % 2]

        def gbody(g, cur=cur):
            v = cur[pl.ds(g * 16, 16)]
            plsc.addupdate_scatter(hist, [v], ones)
        plsc.parallel_loop(0, _GROUPS, 1, unroll=16)(gbody)

    @pl.when(half == 1)
    def _():
        pltpu.sync_copy(hist, shared.at[s // 2])

    plsc.subcore_barrier()

    @pl.when(half == 0)
    def _():
        pltpu.sync_copy(shared.at[s // 2], mbuf)

        @plsc.parallel_loop(0, _NB // 16, 1, unroll=8)
        def _madd(i):
            sl = pl.ds(i * 16, 16)
            hist[sl] = hist[sl] + mbuf[sl]
        pltpu.sync_copy(hist, out_hbm.at[batch])


@functools.lru_cache(maxsize=None)
def _make_hist_sc():
    return pl.kernel(
        _hist_body,
        out_type=jax.ShapeDtypeStruct((_B, _NB), jnp.int32),
        mesh=plsc.VectorSubcoreMesh(core_axis_name="c", subcore_axis_name="s"),
        compiler_params=pltpu.CompilerParams(needs_layout_passes=False),
        scratch_types=[
            pltpu.VMEM((_NB,), jnp.int32),
            pltpu.VMEM((_NB,), jnp.int32),
            pltpu.VMEM_SHARED((8, _NB), jnp.int32),
            pltpu.VMEM((_CH,), jnp.int32),
            pltpu.VMEM((_CH,), jnp.int32),
            pltpu.SemaphoreType.DMA,
            pltpu.SemaphoreType.DMA,
        ],
    )


def kernel(point_cloud):
    x2 = point_cloud.transpose(2, 0, 1).reshape(_R, _N)
    mn, sc = pl.pallas_call(
        _minmax_kernel,
        grid=(_KA,),
        in_specs=[pl.BlockSpec((_R, _CHA), lambda k: (0, k))],
        out_specs=[
            pl.BlockSpec((_R, 128), lambda k: (0, 0)),
            pl.BlockSpec((_R, 128), lambda k: (0, 0)),
        ],
        out_shape=[
            jax.ShapeDtypeStruct((_R, 128), jnp.float32),
            jax.ShapeDtypeStruct((_R, 128), jnp.float32),
        ],
        scratch_shapes=[pltpu.VMEM((_R, 128), jnp.float32)],
    )(x2)

    lin = pl.pallas_call(
        _lin_kernel,
        grid=(_KB,),
        in_specs=[
            pl.BlockSpec((_R, _CHB), lambda k: (0, k)),
            pl.BlockSpec((_R, 128), lambda k: (0, 0)),
            pl.BlockSpec((_R, 128), lambda k: (0, 0)),
        ],
        out_specs=pl.BlockSpec((_B, _CHB), lambda k: (0, k)),
        out_shape=jax.ShapeDtypeStruct((_B, _N), jnp.int32),
    )(x2, mn, sc)

    counts = _make_hist_sc()(lin)
    return lin, counts
